# baseline (speedup 1.0000x reference)
import jax
import jax.numpy as jnp
from jax import lax
from jax.experimental import pallas as pl
from jax.experimental.pallas import tpu as pltpu
from jax.experimental.pallas import tpu_sc as plsc

T = 8192
D = 4096
N = 16384
L = 16
NC = 2
NS = 16
NW = NC * NS
TPW = T // NW
BND = T + L
R = 4
RING = 4
ICH = 2048
AD = 4
UNROLL = 8


def _splat_i(ref, i):
    return plsc.load_gather(ref, [lax.broadcast(i, (L,))])


def _combine_body(expert_hbm, gates_hbm, tok_hbm, out_hbm,
                  gates_v, bnd_v, idxc, rbuf, acc, zrow, rsem, osem):
    wid = lax.axis_index("s") * NC + lax.axis_index("c")
    iota = lax.iota(jnp.int32, L)

    pltpu.sync_copy(gates_hbm, gates_v)

    zi = jnp.zeros((L,), jnp.int32)
    zf = jnp.zeros((L,), jnp.float32)

    @plsc.parallel_loop(0, BND, step=L)
    def init_b(o):
        bnd_v[pl.ds(o, L)] = zi

    @plsc.parallel_loop(0, D, step=L)
    def init_z(o):
        zrow[pl.ds(o, L)] = zf

    for cb in range(N // ICH):
        ext = ICH + L if cb < N // ICH - 1 else ICH
        pltpu.sync_copy(tok_hbm.at[pl.ds(cb * ICH, ext)], idxc.at[pl.ds(0, ext)])

        def trans(p, c, cb=cb, ext=ext):
            posl = p * L + iota
            posg = cb * ICH + posl
            v = idxc[pl.ds(p * L, L)]
            nxt = plsc.load_gather(idxc, [jnp.minimum(posl + 1, ext - 1)])
            is_last = (posg == N - 1) | (v != nxt)
            plsc.store_scatter(bnd_v, [v + 1], posg + 1, mask=is_last)
            return c
        lax.fori_loop(0, ICH // L, trans, 0)

    def ffill(q, carry):
        v = jnp.maximum(plsc.cummax(bnd_v[pl.ds(q * L, L)]), carry)
        bnd_v[pl.ds(q * L, L)] = v
        return lax.broadcast(jnp.max(v), (L,))
    lax.fori_loop(0, BND // L, ffill, lax.broadcast(jnp.int32(0), (L,)))

    t0 = wid * TPW
    lo0 = jnp.max(_splat_i(bnd_v, t0))
    hiw = jnp.max(_splat_i(bnd_v, t0 + TPW))
    b0 = lo0 // R
    blast = (hiw - 1) // R

    def issue(b):
        pltpu.async_copy(expert_hbm.at[pl.ds(b * R, R)],
                         rbuf.at[b % RING], rsem.at[b % RING])

    @pl.when(lo0 < hiw)
    def _():
        for d in range(RING):
            @pl.when(b0 + d <= blast)
            def _(d=d):
                issue(b0 + d)

    def token_body(j, carry):
        lo, pend = carry
        t = t0 + j
        hi = jnp.max(_splat_i(bnd_v, t + 1))
        ab = j & (AD - 1)
        pendb = (pend >> ab) & 1

        @pl.when(hi == lo)
        def _():
            pltpu.sync_copy(zrow, out_hbm.at[t])

        @pl.when(hi > lo)
        def _():
            @pl.when(pendb == 1)
            def _():
                pltpu.make_async_copy(acc.at[ab], out_hbm.at[t],
                                      osem.at[ab]).wait()

            def row_body(r, c):
                blk = r // R
                rb = blk % RING
                slot = r % R

                @pl.when((r == lo0) | (slot == 0))
                def _():
                    @pl.when((blk > b0) & (blk + RING - 1 <= blast))
                    def _():
                        issue(blk + RING - 1)
                    pltpu.make_async_copy(expert_hbm.at[pl.ds(blk * R, R)],
                                          rbuf.at[rb], rsem.at[rb]).wait()

                g = plsc.load_gather(gates_v, [lax.broadcast(r, (L,))])

                @pl.when(r == lo)
                def _():
                    @plsc.parallel_loop(0, D, step=L, unroll=UNROLL)
                    def mul0(o):
                        acc[ab, pl.ds(o, L)] = rbuf[rb, slot, pl.ds(o, L)] * g

                @pl.when(r > lo)
                def _():
                    @plsc.parallel_loop(0, D, step=L, unroll=UNROLL)
                    def mula(o):
                        plsc.addupdate(acc.at[ab, pl.ds(o, L)],
                                       rbuf[rb, slot, pl.ds(o, L)] * g)
                return c
            lax.fori_loop(lo, hi, row_body, 0)
            pltpu.async_copy(acc.at[ab], out_hbm.at[t], osem.at[ab])

        npend = jnp.where(hi > lo, pend | (1 << ab), pend)
        return hi, npend

    lo_end, pend = lax.fori_loop(
        0, TPW, token_body, (lo0, jnp.int32(0)))

    for d in range(AD):
        @pl.when(((pend >> d) & 1) == 1)
        def _(d=d):
            pltpu.make_async_copy(acc.at[d], out_hbm.at[t0],
                                  osem.at[d]).wait()


@jax.jit
def _run(expert_outputs, sorted_gates, tok):
    mesh = plsc.VectorSubcoreMesh(core_axis_name="c", subcore_axis_name="s",
                                  num_cores=NC, num_subcores=NS)
    return pl.kernel(
        _combine_body,
        out_type=jax.ShapeDtypeStruct((T, D), jnp.float32),
        mesh=mesh,
        compiler_params=pltpu.CompilerParams(needs_layout_passes=False),
        scratch_types=[
            pltpu.VMEM((N,), jnp.float32),
            pltpu.VMEM((BND,), jnp.int32),
            pltpu.VMEM((ICH + L,), jnp.int32),
            pltpu.VMEM((RING, R, D), jnp.float32),
            pltpu.VMEM((AD, D), jnp.float32),
            pltpu.VMEM((D,), jnp.float32),
            pltpu.SemaphoreType.DMA((RING,)),
            pltpu.SemaphoreType.DMA((AD,)),
        ],
    )(expert_outputs, sorted_gates, tok)


def kernel(output_buffer, expert_outputs, sorted_gates, token_indices):
    del output_buffer
    return _run(expert_outputs, sorted_gates, token_indices.astype(jnp.int32))


# speedup vs baseline: 1.0608x; 1.0608x over previous
import jax
import jax.numpy as jnp
from jax import lax
from jax.experimental import pallas as pl
from jax.experimental.pallas import tpu as pltpu
from jax.experimental.pallas import tpu_sc as plsc

T = 8192
D = 4096
N = 16384
L = 16
NC = 2
NS = 16
NW = NC * NS
TPW = T // NW
BND = T + L
R = 4
RING = 4
ICH = 2048
AD = 4
UNROLL = 8


def _splat_i(ref, i):
    return plsc.load_gather(ref, [lax.broadcast(i, (L,))])


def _combine_body(expert_hbm, gates_hbm, tok_hbm, out_hbm,
                  gates_v, bnd_v, idxc, rbuf, acc, zrow, rsem, osem, isem):
    wid = lax.axis_index("s") * NC + lax.axis_index("c")
    iota = lax.iota(jnp.int32, L)

    pltpu.sync_copy(gates_hbm, gates_v)

    zi = jnp.zeros((L,), jnp.int32)
    zf = jnp.zeros((L,), jnp.float32)

    @plsc.parallel_loop(0, BND, step=L)
    def init_b(o):
        bnd_v[pl.ds(o, L)] = zi

    @plsc.parallel_loop(0, D, step=L)
    def init_z(o):
        zrow[pl.ds(o, L)] = zf

    NCB = N // ICH

    def chunk_ext(cb):
        return ICH + L if cb < NCB - 1 else ICH

    def issue_chunk(cb):
        pltpu.async_copy(tok_hbm.at[pl.ds(cb * ICH, chunk_ext(cb))],
                         idxc.at[pl.ds((cb % 2) * (ICH + L), chunk_ext(cb))],
                         isem.at[cb % 2])

    issue_chunk(0)
    for cb in range(NCB):
        ext = chunk_ext(cb)
        if cb + 1 < NCB:
            issue_chunk(cb + 1)
        pltpu.make_async_copy(tok_hbm.at[pl.ds(cb * ICH, ext)],
                              idxc.at[pl.ds((cb % 2) * (ICH + L), ext)],
                              isem.at[cb % 2]).wait()

        @plsc.parallel_loop(0, ICH // L, step=1, unroll=4)
        def trans(p, cb=cb, ext=ext):
            posl = p * L + iota
            posg = cb * ICH + posl
            base = (cb % 2) * (ICH + L)
            v = idxc[pl.ds(base + p * L, L)]
            nxt = plsc.load_gather(idxc,
                                   [base + jnp.minimum(posl + 1, ext - 1)])
            is_last = (posg == N - 1) | (v != nxt)
            plsc.store_scatter(bnd_v, [v + 1], posg + 1, mask=is_last)

    def ffill(q, carry):
        v = jnp.maximum(plsc.cummax(bnd_v[pl.ds(q * L, L)]), carry)
        bnd_v[pl.ds(q * L, L)] = v
        return plsc.load_gather(bnd_v, [lax.broadcast(q * L + L - 1, (L,))])
    lax.fori_loop(0, BND // L, ffill, lax.broadcast(jnp.int32(0), (L,)))

    t0 = wid * TPW
    lo0 = jnp.max(_splat_i(bnd_v, t0))
    hiw = jnp.max(_splat_i(bnd_v, t0 + TPW))
    b0 = lo0 // R
    blast = (hiw - 1) // R

    def issue(b):
        pltpu.async_copy(expert_hbm.at[pl.ds(b * R, R)],
                         rbuf.at[b % RING], rsem.at[b % RING])

    @pl.when(lo0 < hiw)
    def _():
        for d in range(RING):
            @pl.when(b0 + d <= blast)
            def _(d=d):
                issue(b0 + d)

    def token_body(j, carry):
        lo, pend = carry
        t = t0 + j
        hi = jnp.max(_splat_i(bnd_v, t + 1))
        ab = j & (AD - 1)
        pendb = (pend >> ab) & 1

        @pl.when(hi == lo)
        def _():
            pltpu.sync_copy(zrow, out_hbm.at[t])

        @pl.when(hi > lo)
        def _():
            @pl.when(pendb == 1)
            def _():
                pltpu.make_async_copy(acc.at[ab], out_hbm.at[t],
                                      osem.at[ab]).wait()

            def row_body(r, c):
                blk = r // R
                rb = blk % RING
                slot = r % R

                @pl.when((r == lo0) | (slot == 0))
                def _():
                    @pl.when((blk > b0) & (blk + RING - 1 <= blast))
                    def _():
                        issue(blk + RING - 1)
                    pltpu.make_async_copy(expert_hbm.at[pl.ds(blk * R, R)],
                                          rbuf.at[rb], rsem.at[rb]).wait()

                g = plsc.load_gather(gates_v, [lax.broadcast(r, (L,))])

                @pl.when(r == lo)
                def _():
                    @plsc.parallel_loop(0, D, step=L, unroll=UNROLL)
                    def mul0(o):
                        acc[ab, pl.ds(o, L)] = rbuf[rb, slot, pl.ds(o, L)] * g

                @pl.when(r > lo)
                def _():
                    @plsc.parallel_loop(0, D, step=L, unroll=UNROLL)
                    def mula(o):
                        plsc.addupdate(acc.at[ab, pl.ds(o, L)],
                                       rbuf[rb, slot, pl.ds(o, L)] * g)
                return c
            lax.fori_loop(lo, hi, row_body, 0)
            pltpu.async_copy(acc.at[ab], out_hbm.at[t], osem.at[ab])

        npend = jnp.where(hi > lo, pend | (1 << ab), pend)
        return hi, npend

    lo_end, pend = lax.fori_loop(
        0, TPW, token_body, (lo0, jnp.int32(0)))

    for d in range(AD):
        @pl.when(((pend >> d) & 1) == 1)
        def _(d=d):
            pltpu.make_async_copy(acc.at[d], out_hbm.at[t0],
                                  osem.at[d]).wait()


@jax.jit
def _run(expert_outputs, sorted_gates, tok):
    mesh = plsc.VectorSubcoreMesh(core_axis_name="c", subcore_axis_name="s",
                                  num_cores=NC, num_subcores=NS)
    return pl.kernel(
        _combine_body,
        out_type=jax.ShapeDtypeStruct((T, D), jnp.float32),
        mesh=mesh,
        compiler_params=pltpu.CompilerParams(needs_layout_passes=False),
        scratch_types=[
            pltpu.VMEM((N,), jnp.float32),
            pltpu.VMEM((BND,), jnp.int32),
            pltpu.VMEM((2 * (ICH + L),), jnp.int32),
            pltpu.VMEM((RING, R, D), jnp.float32),
            pltpu.VMEM((AD, D), jnp.float32),
            pltpu.VMEM((D,), jnp.float32),
            pltpu.SemaphoreType.DMA((RING,)),
            pltpu.SemaphoreType.DMA((AD,)),
            pltpu.SemaphoreType.DMA((2,)),
        ],
    )(expert_outputs, sorted_gates, tok)


def kernel(output_buffer, expert_outputs, sorted_gates, token_indices):
    del output_buffer
    return _run(expert_outputs, sorted_gates, token_indices.astype(jnp.int32))


# speedup vs baseline: 1.0659x; 1.0048x over previous
import jax
import jax.numpy as jnp
from jax import lax
from jax.experimental import pallas as pl
from jax.experimental.pallas import tpu as pltpu
from jax.experimental.pallas import tpu_sc as plsc

T = 8192
D = 4096
N = 16384
L = 16
NC = 2
NS = 16
NW = NC * NS
TPW = T // NW
BND = T + L
R = 8
RING = 2
ICH = 2048
AD = 4
UNROLL = 8


def _splat_i(ref, i):
    return plsc.load_gather(ref, [lax.broadcast(i, (L,))])


def _combine_body(expert_hbm, gates_hbm, tok_hbm, out_hbm,
                  gates_v, bnd_v, idxc, rbuf, acc, zrow, rsem, osem, isem):
    wid = lax.axis_index("s") * NC + lax.axis_index("c")
    iota = lax.iota(jnp.int32, L)

    pltpu.sync_copy(gates_hbm, gates_v)

    zi = jnp.zeros((L,), jnp.int32)
    zf = jnp.zeros((L,), jnp.float32)

    @plsc.parallel_loop(0, BND, step=L)
    def init_b(o):
        bnd_v[pl.ds(o, L)] = zi

    @plsc.parallel_loop(0, D, step=L)
    def init_z(o):
        zrow[pl.ds(o, L)] = zf

    NCB = N // ICH

    def chunk_ext(cb):
        return ICH + L if cb < NCB - 1 else ICH

    def issue_chunk(cb):
        pltpu.async_copy(tok_hbm.at[pl.ds(cb * ICH, chunk_ext(cb))],
                         idxc.at[pl.ds((cb % 2) * (ICH + L), chunk_ext(cb))],
                         isem.at[cb % 2])

    issue_chunk(0)
    for cb in range(NCB):
        ext = chunk_ext(cb)
        if cb + 1 < NCB:
            issue_chunk(cb + 1)
        pltpu.make_async_copy(tok_hbm.at[pl.ds(cb * ICH, ext)],
                              idxc.at[pl.ds((cb % 2) * (ICH + L), ext)],
                              isem.at[cb % 2]).wait()

        @plsc.parallel_loop(0, ICH // L, step=1, unroll=4)
        def trans(p, cb=cb, ext=ext):
            posl = p * L + iota
            posg = cb * ICH + posl
            base = (cb % 2) * (ICH + L)
            v = idxc[pl.ds(base + p * L, L)]
            nxt = plsc.load_gather(idxc,
                                   [base + jnp.minimum(posl + 1, ext - 1)])
            is_last = (posg == N - 1) | (v != nxt)
            plsc.store_scatter(bnd_v, [v + 1], posg + 1, mask=is_last)

    def ffill(q, carry):
        v = jnp.maximum(plsc.cummax(bnd_v[pl.ds(q * L, L)]), carry)
        bnd_v[pl.ds(q * L, L)] = v
        return plsc.load_gather(bnd_v, [lax.broadcast(q * L + L - 1, (L,))])
    lax.fori_loop(0, BND // L, ffill, lax.broadcast(jnp.int32(0), (L,)))

    t0 = wid * TPW
    lo0 = jnp.max(_splat_i(bnd_v, t0))
    hiw = jnp.max(_splat_i(bnd_v, t0 + TPW))
    b0 = lo0 // R
    blast = (hiw - 1) // R

    def issue(b):
        pltpu.async_copy(expert_hbm.at[pl.ds(b * R, R)],
                         rbuf.at[b % RING], rsem.at[b % RING])

    @pl.when(lo0 < hiw)
    def _():
        for d in range(RING):
            @pl.when(b0 + d <= blast)
            def _(d=d):
                issue(b0 + d)

    def token_body(j, carry):
        lo, pend = carry
        t = t0 + j
        hi = jnp.max(_splat_i(bnd_v, t + 1))
        ab = j & (AD - 1)
        pendb = (pend >> ab) & 1

        @pl.when(hi == lo)
        def _():
            pltpu.sync_copy(zrow, out_hbm.at[t])

        @pl.when(hi > lo)
        def _():
            @pl.when(pendb == 1)
            def _():
                pltpu.make_async_copy(acc.at[ab], out_hbm.at[t],
                                      osem.at[ab]).wait()

            def row_body(r, c):
                blk = r // R
                rb = blk % RING
                slot = r % R

                @pl.when((r == lo0) | (slot == 0))
                def _():
                    @pl.when((blk > b0) & (blk + RING - 1 <= blast))
                    def _():
                        issue(blk + RING - 1)
                    pltpu.make_async_copy(expert_hbm.at[pl.ds(blk * R, R)],
                                          rbuf.at[rb], rsem.at[rb]).wait()

                g = plsc.load_gather(gates_v, [lax.broadcast(r, (L,))])

                @pl.when(r == lo)
                def _():
                    @plsc.parallel_loop(0, D, step=L, unroll=UNROLL)
                    def mul0(o):
                        acc[ab, pl.ds(o, L)] = rbuf[rb, slot, pl.ds(o, L)] * g

                @pl.when(r > lo)
                def _():
                    @plsc.parallel_loop(0, D, step=L, unroll=UNROLL)
                    def mula(o):
                        plsc.addupdate(acc.at[ab, pl.ds(o, L)],
                                       rbuf[rb, slot, pl.ds(o, L)] * g)
                return c
            lax.fori_loop(lo, hi, row_body, 0)
            pltpu.async_copy(acc.at[ab], out_hbm.at[t], osem.at[ab])

        npend = jnp.where(hi > lo, pend | (1 << ab), pend)
        return hi, npend

    lo_end, pend = lax.fori_loop(
        0, TPW, token_body, (lo0, jnp.int32(0)))

    for d in range(AD):
        @pl.when(((pend >> d) & 1) == 1)
        def _(d=d):
            pltpu.make_async_copy(acc.at[d], out_hbm.at[t0],
                                  osem.at[d]).wait()


@jax.jit
def _run(expert_outputs, sorted_gates, tok):
    mesh = plsc.VectorSubcoreMesh(core_axis_name="c", subcore_axis_name="s",
                                  num_cores=NC, num_subcores=NS)
    return pl.kernel(
        _combine_body,
        out_type=jax.ShapeDtypeStruct((T, D), jnp.float32),
        mesh=mesh,
        compiler_params=pltpu.CompilerParams(needs_layout_passes=False),
        scratch_types=[
            pltpu.VMEM((N,), jnp.float32),
            pltpu.VMEM((BND,), jnp.int32),
            pltpu.VMEM((2 * (ICH + L),), jnp.int32),
            pltpu.VMEM((RING, R, D), jnp.float32),
            pltpu.VMEM((AD, D), jnp.float32),
            pltpu.VMEM((D,), jnp.float32),
            pltpu.SemaphoreType.DMA((RING,)),
            pltpu.SemaphoreType.DMA((AD,)),
            pltpu.SemaphoreType.DMA((2,)),
        ],
    )(expert_outputs, sorted_gates, tok)


def kernel(output_buffer, expert_outputs, sorted_gates, token_indices):
    del output_buffer
    return _run(expert_outputs, sorted_gates, token_indices.astype(jnp.int32))
